# baseline (speedup 1.0000x reference)
import functools

import jax
import jax.numpy as jnp
from jax import lax
from jax.experimental import pallas as pl
from jax.experimental.pallas import tpu as pltpu
from jax.experimental.pallas import tpu_sc as plsc

VOCAB = 100000
DIM = 64
BATCH = 1024
CTX = 20

NUM_CORES = 2
NUM_SUBCORES = 16
NW = NUM_CORES * NUM_SUBCORES
PASSES = DIM // NW
LANES = 16
B_CHUNKS = BATCH // LANES

_mesh = plsc.VectorSubcoreMesh(core_axis_name="c", subcore_axis_name="s")


@functools.partial(
    pl.kernel,
    mesh=_mesh,
    out_type=jax.ShapeDtypeStruct((DIM, BATCH), jnp.float32),
    scratch_types=[
        pltpu.VMEM((CTX, 8, 128), jnp.int32),
        pltpu.VMEM((VOCAB,), jnp.float32),
        pltpu.VMEM((PASSES, BATCH), jnp.float32),
        pltpu.SemaphoreType.DMA,
    ],
    compiler_params=pltpu.CompilerParams(
        use_tc_tiling_on_sc=True, needs_layout_passes=False
    ),
)
def _pool_sc(xt_hbm, embt_hbm, ht_hbm, xt_v, row_v, hrow_v, sem):
    wid = lax.axis_index("s") * NUM_CORES + lax.axis_index("c")
    pltpu.sync_copy(xt_hbm, xt_v)
    scale = jnp.full((LANES,), 1.0 / CTX, jnp.float32)

    for p in range(PASSES):
        d = wid + NW * p
        pltpu.sync_copy(embt_hbm.at[d], row_v)

        def body(i, p=p):
            s = i // 8
            l0 = (i % 8) * LANES
            acc = jnp.zeros((LANES,), jnp.float32)
            for c in range(CTX):
                idx = xt_v[c, s, pl.ds(l0, LANES)]
                acc = acc + plsc.load_gather(row_v, [idx])
            hrow_v[p, pl.ds(i * LANES, LANES)] = acc * scale

        plsc.parallel_loop(0, B_CHUNKS, 1, unroll=8)(body)

    for p in range(PASSES):
        d = wid + NW * p
        pltpu.sync_copy(hrow_v.at[p], ht_hbm.at[d])


V_TILE = 4096
V_GRID = (VOCAB + V_TILE - 1) // V_TILE


def _proj_body(wt_ref, ht_ref, ot_ref):
    ot_ref[...] = lax.dot_general(
        wt_ref[...],
        ht_ref[...],
        (((0,), (0,)), ((), ())),
        preferred_element_type=jnp.float32,
    )


def _project_t(wt, ht):
    return pl.pallas_call(
        _proj_body,
        grid=(V_GRID,),
        in_specs=[
            pl.BlockSpec((DIM, V_TILE), lambda i: (0, i)),
            pl.BlockSpec((DIM, BATCH), lambda i: (0, 0)),
        ],
        out_specs=pl.BlockSpec((V_TILE, BATCH), lambda i: (i, 0)),
        out_shape=jax.ShapeDtypeStruct((VOCAB, BATCH), jnp.float32),
        compiler_params=pltpu.CompilerParams(
            dimension_semantics=("arbitrary",),
        ),
    )(wt, ht)


def kernel(x, emb, W):
    xt = jnp.swapaxes(x, 0, 1).astype(jnp.int32).reshape(CTX, 8, 128)
    embt = jnp.swapaxes(emb, 0, 1)
    ht = _pool_sc(xt, embt)
    out_t = _project_t(jnp.swapaxes(W, 0, 1), ht)
    return jnp.swapaxes(out_t, 0, 1)


# speedup vs baseline: 1.0082x; 1.0082x over previous
import functools

import jax
import jax.numpy as jnp
from jax import lax
from jax.experimental import pallas as pl
from jax.experimental.pallas import tpu as pltpu
from jax.experimental.pallas import tpu_sc as plsc

VOCAB = 100000
DIM = 64
BATCH = 1024
CTX = 20

NUM_CORES = 2
NUM_SUBCORES = 16
NW = NUM_CORES * NUM_SUBCORES
PASSES = DIM // NW
LANES = 16
B_CHUNKS = BATCH // LANES

_mesh = plsc.VectorSubcoreMesh(core_axis_name="c", subcore_axis_name="s")


@functools.partial(
    pl.kernel,
    mesh=_mesh,
    out_type=jax.ShapeDtypeStruct((DIM, BATCH), jnp.float32),
    scratch_types=[
        pltpu.VMEM((CTX, 8, 128), jnp.int32),
        pltpu.VMEM((VOCAB,), jnp.float32),
        pltpu.VMEM((PASSES, BATCH), jnp.float32),
        pltpu.SemaphoreType.DMA,
    ],
    compiler_params=pltpu.CompilerParams(
        use_tc_tiling_on_sc=True, needs_layout_passes=False
    ),
)
def _pool_sc(xt_hbm, embt_hbm, ht_hbm, xt_v, row_v, hrow_v, sem):
    wid = lax.axis_index("s") * NUM_CORES + lax.axis_index("c")
    pltpu.sync_copy(xt_hbm, xt_v)
    scale = jnp.full((LANES,), 1.0 / CTX, jnp.float32)

    for p in range(PASSES):
        d = wid + NW * p
        pltpu.sync_copy(embt_hbm.at[d], row_v)

        def body(i, p=p):
            s = i // 8
            l0 = (i % 8) * LANES
            acc = jnp.zeros((LANES,), jnp.float32)
            for c in range(CTX):
                idx = xt_v[c, s, pl.ds(l0, LANES)]
                acc = acc + plsc.load_gather(row_v, [idx])
            hrow_v[p, pl.ds(i * LANES, LANES)] = acc * scale

        plsc.parallel_loop(0, B_CHUNKS, 1, unroll=4)(body)

    for p in range(PASSES):
        d = wid + NW * p
        pltpu.sync_copy(hrow_v.at[p], ht_hbm.at[d])


V_TILE = 4096
V_GRID = (VOCAB + V_TILE - 1) // V_TILE


def _proj_body(wt_ref, ht_ref, ot_ref):
    ot_ref[...] = lax.dot_general(
        wt_ref[...],
        ht_ref[...],
        (((0,), (0,)), ((), ())),
        preferred_element_type=jnp.float32,
    )


def _project_t(wt, ht):
    return pl.pallas_call(
        _proj_body,
        grid=(V_GRID,),
        in_specs=[
            pl.BlockSpec((DIM, V_TILE), lambda i: (0, i)),
            pl.BlockSpec((DIM, BATCH), lambda i: (0, 0)),
        ],
        out_specs=pl.BlockSpec((V_TILE, BATCH), lambda i: (i, 0)),
        out_shape=jax.ShapeDtypeStruct((VOCAB, BATCH), jnp.float32),
        compiler_params=pltpu.CompilerParams(
            dimension_semantics=("arbitrary",),
        ),
    )(wt, ht)


def kernel(x, emb, W):
    xt = jnp.swapaxes(x, 0, 1).astype(jnp.int32).reshape(CTX, 8, 128)
    embt = jnp.swapaxes(emb, 0, 1)
    ht = _pool_sc(xt, embt)
    out_t = _project_t(jnp.swapaxes(W, 0, 1), ht)
    return jnp.swapaxes(out_t, 0, 1)


# speedup vs baseline: 1.0143x; 1.0060x over previous
import functools

import jax
import jax.numpy as jnp
from jax import lax
from jax.experimental import pallas as pl
from jax.experimental.pallas import tpu as pltpu
from jax.experimental.pallas import tpu_sc as plsc

VOCAB = 100000
DIM = 64
BATCH = 1024
CTX = 20

NUM_CORES = 2
NUM_SUBCORES = 16
NW = NUM_CORES * NUM_SUBCORES
PASSES = DIM // NW
LANES = 16
B_CHUNKS = BATCH // LANES

_mesh = plsc.VectorSubcoreMesh(core_axis_name="c", subcore_axis_name="s")


@functools.partial(
    pl.kernel,
    mesh=_mesh,
    out_type=jax.ShapeDtypeStruct((DIM, BATCH), jnp.float32),
    scratch_types=[
        pltpu.VMEM((CTX, 8, 128), jnp.int32),
        pltpu.VMEM((VOCAB,), jnp.float32),
        pltpu.VMEM((PASSES, BATCH), jnp.float32),
        pltpu.SemaphoreType.DMA,
        pltpu.SemaphoreType.DMA,
    ],
    compiler_params=pltpu.CompilerParams(
        use_tc_tiling_on_sc=True, needs_layout_passes=False
    ),
)
def _pool_sc(xt_hbm, embt_hbm, ht_hbm, xt_v, row_v, hrow_v, sem, semx):
    wid = lax.axis_index("s") * NUM_CORES + lax.axis_index("c")
    cx = pltpu.async_copy(xt_hbm, xt_v, semx)
    c0 = pltpu.async_copy(embt_hbm.at[wid], row_v, sem)
    cx.wait()
    scale = jnp.full((LANES,), 1.0 / CTX, jnp.float32)

    for p in range(PASSES):
        d = wid + NW * p
        if p == 0:
            c0.wait()
        else:
            pltpu.sync_copy(embt_hbm.at[d], row_v)

        def body(i, p=p):
            s = i // 8
            l0 = (i % 8) * LANES
            acc = jnp.zeros((LANES,), jnp.float32)
            for c in range(CTX):
                idx = xt_v[c, s, pl.ds(l0, LANES)]
                acc = acc + plsc.load_gather(row_v, [idx])
            hrow_v[p, pl.ds(i * LANES, LANES)] = acc * scale

        plsc.parallel_loop(0, B_CHUNKS, 1, unroll=4)(body)

    for p in range(PASSES):
        d = wid + NW * p
        pltpu.sync_copy(hrow_v.at[p], ht_hbm.at[d])


V_TILE = 4096
V_GRID = (VOCAB + V_TILE - 1) // V_TILE


def _proj_body(wt_ref, ht_ref, ot_ref):
    ot_ref[...] = lax.dot_general(
        wt_ref[...],
        ht_ref[...],
        (((0,), (0,)), ((), ())),
        preferred_element_type=jnp.float32,
    )


def _project_t(wt, ht):
    return pl.pallas_call(
        _proj_body,
        grid=(V_GRID,),
        in_specs=[
            pl.BlockSpec((DIM, V_TILE), lambda i: (0, i)),
            pl.BlockSpec((DIM, BATCH), lambda i: (0, 0)),
        ],
        out_specs=pl.BlockSpec((V_TILE, BATCH), lambda i: (i, 0)),
        out_shape=jax.ShapeDtypeStruct((VOCAB, BATCH), jnp.float32),
        compiler_params=pltpu.CompilerParams(
            dimension_semantics=("arbitrary",),
        ),
    )(wt, ht)


def kernel(x, emb, W):
    xt = jnp.swapaxes(x, 0, 1).astype(jnp.int32).reshape(CTX, 8, 128)
    embt = jnp.swapaxes(emb, 0, 1)
    ht = _pool_sc(xt, embt)
    out_t = _project_t(jnp.swapaxes(W, 0, 1), ht)
    return jnp.swapaxes(out_t, 0, 1)
